# Initial kernel scaffold; baseline (speedup 1.0000x reference)
import jax
import jax.numpy as jnp
from jax.experimental import pallas as pl


def kernel(a, b, table, W_proj, b_proj, W1, b1, W2, b2):
    raise NotImplementedError("write your pallas kernel here")


# speedup vs baseline: 2.6289x; 2.6289x over previous
import functools

import jax
import jax.numpy as jnp
from jax import lax
from jax.experimental import pallas as pl
from jax.experimental.pallas import tpu as pltpu
from jax.experimental.pallas import tpu_sc as plsc

B = 16384
L = 50
EMB = 64
PROJ = 128

_IDXW = 128
_GF = 8
_CHUNK = _IDXW * _GF


def _make_sc_gather(n_tokens: int):
    info = plsc.get_sparse_core_info()
    nw = info.num_cores * info.num_subcores
    per_w = n_tokens // nw
    assert n_tokens % nw == 0 and per_w % _CHUNK == 0
    n_groups = per_w // _CHUNK
    mesh = plsc.VectorSubcoreMesh(core_axis_name="c", subcore_axis_name="s")

    @functools.partial(
        pl.kernel,
        mesh=mesh,
        out_type=jax.ShapeDtypeStruct((n_tokens, EMB), jnp.float32),
        scratch_types=[
            pltpu.VMEM((_GF, _IDXW), jnp.int32),
            pltpu.VMEM((_CHUNK, EMB), jnp.float32),
            pltpu.SemaphoreType.DMA,
        ],
        compiler_params=pltpu.CompilerParams(use_tc_tiling_on_sc=False),
    )
    def gather_k(table_hbm, idx_hbm, out_hbm, idx_v, rows_v, sem):
        wid = lax.axis_index("s") * info.num_cores + lax.axis_index("c")
        base = wid * per_w

        def body(g, carry):
            off = base + g * _CHUNK
            pltpu.sync_copy(idx_hbm.at[pl.ds(off // _IDXW, _GF)], idx_v)
            copies = [
                pltpu.async_copy(
                    table_hbm.at[idx_v.at[j]],
                    rows_v.at[pl.ds(j * _IDXW, _IDXW)],
                    sem,
                )
                for j in range(_GF)
            ]
            for c in copies:
                c.wait()
            pltpu.sync_copy(rows_v, out_hbm.at[pl.ds(off, _CHUNK)])
            return carry

        lax.fori_loop(0, n_groups, body, 0)

    return gather_k


_BB = 256
_NBLK = B // _BB
_PR = _BB * L // 2


def _dense_kernel(ea_ref, eb_ref, wp2_ref, bp2_ref, w1a_ref, w1b_ref, b1_ref,
                  w2t_ref, b2_ref, out_ref):
    wp2 = wp2_ref[...].astype(jnp.bfloat16)
    bp2 = bp2_ref[...]

    def tower(e_ref):
        e = e_ref[...].astype(jnp.bfloat16)
        p = jnp.dot(e, wp2, preferred_element_type=jnp.float32) + bp2
        p = jnp.maximum(p, 0.0)
        s = jnp.sum(p.reshape(_BB, L // 2, 2 * PROJ), axis=1)
        return (s[:, :PROJ] + s[:, PROJ:]) * (1.0 / L)

    ma = tower(ea_ref)
    mb = tower(eb_ref)
    h = (jnp.dot(ma.astype(jnp.bfloat16), w1a_ref[...].astype(jnp.bfloat16),
                 preferred_element_type=jnp.float32)
         + jnp.dot(mb.astype(jnp.bfloat16), w1b_ref[...].astype(jnp.bfloat16),
                   preferred_element_type=jnp.float32)
         + b1_ref[...])
    h = jnp.maximum(h, 0.0)
    o = jnp.sum(h * w2t_ref[...], axis=1) + b2_ref[0]
    out_ref[0, 0, :] = o


def kernel(a, b, table, W_proj, b_proj, W1, b1, W2, b2):
    n_tokens = 2 * B * L
    idx = jnp.concatenate([a.reshape(-1), b.reshape(-1)])
    idx2 = idx.reshape(n_tokens // _IDXW, _IDXW)
    gathered = _make_sc_gather(n_tokens)(table, idx2)
    pairs = gathered.reshape(n_tokens // 2, 2 * EMB)

    wp2 = jnp.zeros((2 * EMB, 2 * PROJ), jnp.float32)
    wp2 = wp2.at[:EMB, :PROJ].set(W_proj).at[EMB:, PROJ:].set(W_proj)
    bp2 = jnp.concatenate([b_proj, b_proj]).reshape(1, 2 * PROJ)

    out = pl.pallas_call(
        _dense_kernel,
        grid=(_NBLK,),
        in_specs=[
            pl.BlockSpec((_PR, 2 * EMB), lambda i: (i, 0)),
            pl.BlockSpec((_PR, 2 * EMB), lambda i: (i + _NBLK, 0)),
            pl.BlockSpec((2 * EMB, 2 * PROJ), lambda i: (0, 0)),
            pl.BlockSpec((1, 2 * PROJ), lambda i: (0, 0)),
            pl.BlockSpec((PROJ, PROJ), lambda i: (0, 0)),
            pl.BlockSpec((PROJ, PROJ), lambda i: (0, 0)),
            pl.BlockSpec((1, PROJ), lambda i: (0, 0)),
            pl.BlockSpec((1, PROJ), lambda i: (0, 0)),
            pl.BlockSpec(memory_space=pltpu.SMEM),
        ],
        out_specs=pl.BlockSpec((1, 1, _BB), lambda i: (i, 0, 0)),
        out_shape=jax.ShapeDtypeStruct((_NBLK, 1, _BB), jnp.float32),
    )(
        pairs, pairs,
        wp2, bp2,
        W1[:PROJ], W1[PROJ:], b1.reshape(1, PROJ),
        W2.reshape(1, PROJ), b2,
    )
    return out.reshape(B)


# speedup vs baseline: 3.6306x; 1.3810x over previous
import functools

import jax
import jax.numpy as jnp
from jax import lax
from jax.experimental import pallas as pl
from jax.experimental.pallas import tpu as pltpu
from jax.experimental.pallas import tpu_sc as plsc

B = 16384
L = 50
EMB = 64
PROJ = 128

_IDXW = 128
_GF = 8
_CHUNK = _IDXW * _GF


def _make_sc_gather(n_tokens: int):
    info = plsc.get_sparse_core_info()
    nw = info.num_cores * info.num_subcores
    per_w = n_tokens // nw
    assert n_tokens % nw == 0 and per_w % _CHUNK == 0
    n_groups = per_w // _CHUNK
    mesh = plsc.VectorSubcoreMesh(core_axis_name="c", subcore_axis_name="s")

    @functools.partial(
        pl.kernel,
        mesh=mesh,
        out_type=jax.ShapeDtypeStruct((n_tokens, EMB), jnp.float32),
        scratch_types=[
            pltpu.VMEM((_GF, _IDXW), jnp.int32),
            pltpu.VMEM((_CHUNK, EMB), jnp.float32),
            pltpu.SemaphoreType.DMA,
        ],
        compiler_params=pltpu.CompilerParams(use_tc_tiling_on_sc=False),
    )
    def gather_k(table_hbm, idx_hbm, out_hbm, idx_v, rows_v, sem):
        wid = lax.axis_index("s") * info.num_cores + lax.axis_index("c")
        base = wid * per_w

        def body(g, carry):
            off = base + g * _CHUNK
            pltpu.sync_copy(idx_hbm.at[pl.ds(off // _IDXW, _GF)], idx_v)
            copies = [
                pltpu.async_copy(
                    table_hbm.at[idx_v.at[j]],
                    rows_v.at[pl.ds(j * _IDXW, _IDXW)],
                    sem,
                )
                for j in range(_GF)
            ]
            for c in copies:
                c.wait()
            pltpu.sync_copy(rows_v, out_hbm.at[pl.ds(off, _CHUNK)])
            return carry

        lax.fori_loop(0, n_groups, body, 0)

    return gather_k


_BP = 128
_NBLK = B // (2 * _BP)


def _dense_kernel(ea_ref, eb_ref, wp2_ref, bp2_ref, w1a_ref, w1b_ref, b1_ref,
                  w2t_ref, b2_ref, out_ref):
    wp2 = wp2_ref[...].astype(jnp.bfloat16)
    bp2 = bp2_ref[...]
    inv_l = 1.0 / L

    def tower(e_ref):
        e = e_ref[...].reshape(L * _BP, 2 * EMB).astype(jnp.bfloat16)
        p = jnp.dot(e, wp2, preferred_element_type=jnp.float32) + bp2
        p = jnp.maximum(p, 0.0)
        s = jnp.sum(p.reshape(L, _BP, 2 * PROJ), axis=0)
        return s[:, :PROJ] * inv_l, s[:, PROJ:] * inv_l

    maE, maO = tower(ea_ref)
    mbE, mbO = tower(eb_ref)
    w1a = w1a_ref[...].astype(jnp.bfloat16)
    w1b = w1b_ref[...].astype(jnp.bfloat16)

    def head(ma, mb, par):
        h = (jnp.dot(ma.astype(jnp.bfloat16), w1a,
                     preferred_element_type=jnp.float32)
             + jnp.dot(mb.astype(jnp.bfloat16), w1b,
                       preferred_element_type=jnp.float32)
             + b1_ref[...])
        h = jnp.maximum(h, 0.0)
        o = jnp.sum(h * w2t_ref[...], axis=1) + b2_ref[0]
        out_ref[0, par, :] = o

    head(maE, mbE, 0)
    head(maO, mbO, 1)


def kernel(a, b, table, W_proj, b_proj, W1, b1, W2, b2):
    n_tokens = 2 * B * L
    idx = jnp.concatenate([a.T.reshape(-1), b.T.reshape(-1)])
    idx2 = idx.reshape(n_tokens // _IDXW, _IDXW)
    gathered = _make_sc_gather(n_tokens)(table, idx2)
    ev = gathered.reshape(2 * L, B // 2, 2 * EMB)

    wp2 = jnp.zeros((2 * EMB, 2 * PROJ), jnp.float32)
    wp2 = wp2.at[:EMB, :PROJ].set(W_proj).at[EMB:, PROJ:].set(W_proj)
    bp2 = jnp.concatenate([b_proj, b_proj]).reshape(1, 2 * PROJ)

    out = pl.pallas_call(
        _dense_kernel,
        grid=(_NBLK,),
        in_specs=[
            pl.BlockSpec((L, _BP, 2 * EMB), lambda i: (0, i, 0)),
            pl.BlockSpec((L, _BP, 2 * EMB), lambda i: (1, i, 0)),
            pl.BlockSpec((2 * EMB, 2 * PROJ), lambda i: (0, 0)),
            pl.BlockSpec((1, 2 * PROJ), lambda i: (0, 0)),
            pl.BlockSpec((PROJ, PROJ), lambda i: (0, 0)),
            pl.BlockSpec((PROJ, PROJ), lambda i: (0, 0)),
            pl.BlockSpec((1, PROJ), lambda i: (0, 0)),
            pl.BlockSpec((1, PROJ), lambda i: (0, 0)),
            pl.BlockSpec(memory_space=pltpu.SMEM),
        ],
        out_specs=pl.BlockSpec((1, 2, _BP), lambda i: (i, 0, 0)),
        out_shape=jax.ShapeDtypeStruct((_NBLK, 2, _BP), jnp.float32),
    )(
        ev, ev,
        wp2, bp2,
        W1[:PROJ], W1[PROJ:], b1.reshape(1, PROJ),
        W2.reshape(1, PROJ), b2,
    )
    return out.transpose(0, 2, 1).reshape(B)


# speedup vs baseline: 5.1061x; 1.4064x over previous
import functools

import jax
import jax.numpy as jnp
from jax import lax
from jax.experimental import pallas as pl
from jax.experimental.pallas import tpu as pltpu
from jax.experimental.pallas import tpu_sc as plsc

B = 16384
L = 50
EMB = 64
PROJ = 128

_IDXW = 128
_GF = 8
_CHUNK = _IDXW * _GF


def _make_sc_gather(n_tokens: int):
    info = plsc.get_sparse_core_info()
    nw = info.num_cores * info.num_subcores
    per_w = n_tokens // nw
    assert n_tokens % nw == 0 and per_w % _CHUNK == 0
    n_groups = per_w // _CHUNK
    mesh = plsc.VectorSubcoreMesh(core_axis_name="c", subcore_axis_name="s")

    @functools.partial(
        pl.kernel,
        mesh=mesh,
        out_type=jax.ShapeDtypeStruct((n_tokens, EMB), jnp.float32),
        scratch_types=[
            pltpu.VMEM((_GF, _IDXW), jnp.int32),
            pltpu.VMEM((_CHUNK, EMB), jnp.float32),
            pltpu.SemaphoreType.DMA,
        ],
        compiler_params=pltpu.CompilerParams(use_tc_tiling_on_sc=False),
    )
    def gather_k(table_hbm, idx_hbm, out_hbm, idx_v, rows_v, sem):
        wid = lax.axis_index("s") * info.num_cores + lax.axis_index("c")
        base = wid * per_w

        def body(g, carry):
            off = base + g * _CHUNK
            pltpu.sync_copy(idx_hbm.at[pl.ds(off // _IDXW, _GF)], idx_v)
            copies = [
                pltpu.async_copy(
                    table_hbm.at[idx_v.at[j]],
                    rows_v.at[pl.ds(j * _IDXW, _IDXW)],
                    sem,
                )
                for j in range(_GF)
            ]
            for c in copies:
                c.wait()
            pltpu.sync_copy(rows_v, out_hbm.at[pl.ds(off, _CHUNK)])
            return carry

        lax.fori_loop(0, n_groups, body, 0)

    return gather_k


_TCOLS = 8192
_TH = _TCOLS // 2
VOCAB = 1000000
_NTBLK = (VOCAB + _TCOLS - 1) // _TCOLS
VPAD = _NTBLK * _TCOLS


def _repack_kernel(t_ref, o_ref):
    x = t_ref[...]
    o_ref[...] = jnp.concatenate([x[:, :_TH].T, x[:, _TH:].T], axis=1)


def _repack_table(tT):
    return pl.pallas_call(
        _repack_kernel,
        grid=(_NTBLK,),
        in_specs=[pl.BlockSpec((EMB, _TCOLS), lambda i: (0, i))],
        out_specs=pl.BlockSpec((_TH, 2 * EMB), lambda i: (i, 0)),
        out_shape=jax.ShapeDtypeStruct((VPAD // 2, 2 * EMB), jnp.float32),
    )(tT)


def _remap_idx(idx):
    blk = idx & (_TCOLS - 1)
    return (idx - blk) + ((blk & (_TH - 1)) << 1) + (blk >> 12)


_BP = 128
_NBLK = B // (2 * _BP)


def _dense_kernel(ea_ref, eb_ref, wp2_ref, bp2_ref, w1a_ref, w1b_ref, b1_ref,
                  w2t_ref, b2_ref, out_ref):
    wp2 = wp2_ref[...].astype(jnp.bfloat16)
    bp2 = bp2_ref[...]
    inv_l = 1.0 / L

    def tower(e_ref):
        e = e_ref[...].reshape(L * _BP, 2 * EMB).astype(jnp.bfloat16)
        p = jnp.dot(e, wp2, preferred_element_type=jnp.float32) + bp2
        p = jnp.maximum(p, 0.0)
        s = jnp.sum(p.reshape(L, _BP, 2 * PROJ), axis=0)
        return s[:, :PROJ] * inv_l, s[:, PROJ:] * inv_l

    maE, maO = tower(ea_ref)
    mbE, mbO = tower(eb_ref)
    w1a = w1a_ref[...].astype(jnp.bfloat16)
    w1b = w1b_ref[...].astype(jnp.bfloat16)

    def head(ma, mb, par):
        h = (jnp.dot(ma.astype(jnp.bfloat16), w1a,
                     preferred_element_type=jnp.float32)
             + jnp.dot(mb.astype(jnp.bfloat16), w1b,
                       preferred_element_type=jnp.float32)
             + b1_ref[...])
        h = jnp.maximum(h, 0.0)
        o = jnp.sum(h * w2t_ref[...], axis=1) + b2_ref[0]
        out_ref[0, par, :] = o

    head(maE, mbE, 0)
    head(maO, mbO, 1)


def kernel(a, b, table, W_proj, b_proj, W1, b1, W2, b2):
    n_tokens = 2 * B * L
    idx = _remap_idx(jnp.concatenate([a.T.reshape(-1), b.T.reshape(-1)]))
    idx2 = idx.reshape(n_tokens // _IDXW, _IDXW)
    table_lin = _repack_table(table.T).reshape(VPAD, EMB)
    gathered = _make_sc_gather(n_tokens)(table_lin, idx2)
    ev = gathered.reshape(2 * L, B // 2, 2 * EMB)

    wp2 = jnp.zeros((2 * EMB, 2 * PROJ), jnp.float32)
    wp2 = wp2.at[:EMB, :PROJ].set(W_proj).at[EMB:, PROJ:].set(W_proj)
    bp2 = jnp.concatenate([b_proj, b_proj]).reshape(1, 2 * PROJ)

    out = pl.pallas_call(
        _dense_kernel,
        grid=(_NBLK,),
        in_specs=[
            pl.BlockSpec((L, _BP, 2 * EMB), lambda i: (0, i, 0)),
            pl.BlockSpec((L, _BP, 2 * EMB), lambda i: (1, i, 0)),
            pl.BlockSpec((2 * EMB, 2 * PROJ), lambda i: (0, 0)),
            pl.BlockSpec((1, 2 * PROJ), lambda i: (0, 0)),
            pl.BlockSpec((PROJ, PROJ), lambda i: (0, 0)),
            pl.BlockSpec((PROJ, PROJ), lambda i: (0, 0)),
            pl.BlockSpec((1, PROJ), lambda i: (0, 0)),
            pl.BlockSpec((1, PROJ), lambda i: (0, 0)),
            pl.BlockSpec(memory_space=pltpu.SMEM),
        ],
        out_specs=pl.BlockSpec((1, 2, _BP), lambda i: (i, 0, 0)),
        out_shape=jax.ShapeDtypeStruct((_NBLK, 2, _BP), jnp.float32),
    )(
        ev, ev,
        wp2, bp2,
        W1[:PROJ], W1[PROJ:], b1.reshape(1, PROJ),
        W2.reshape(1, PROJ), b2,
    )
    return out.transpose(0, 2, 1).reshape(B)


# speedup vs baseline: 5.2105x; 1.0205x over previous
import functools

import jax
import jax.numpy as jnp
from jax import lax
from jax.experimental import pallas as pl
from jax.experimental.pallas import tpu as pltpu
from jax.experimental.pallas import tpu_sc as plsc

B = 16384
L = 50
EMB = 64
PROJ = 128

_IDXW = 128
_GF = 4
_CHUNK = _IDXW * _GF


def _make_sc_gather(n_tokens: int):
    info = plsc.get_sparse_core_info()
    nw = info.num_cores * info.num_subcores
    per_w = n_tokens // nw
    assert n_tokens % nw == 0 and per_w % (2 * _CHUNK) == 0
    n_pairs = per_w // (2 * _CHUNK)
    mesh = plsc.VectorSubcoreMesh(core_axis_name="c", subcore_axis_name="s")

    @functools.partial(
        pl.kernel,
        mesh=mesh,
        out_type=jax.ShapeDtypeStruct((n_tokens, EMB), jnp.float32),
        scratch_types=[
            pltpu.VMEM((2, _GF, _IDXW), jnp.int32),
            pltpu.VMEM((2, _CHUNK, EMB), jnp.float32),
            pltpu.SemaphoreType.DMA,
            pltpu.SemaphoreType.DMA,
        ],
        compiler_params=pltpu.CompilerParams(use_tc_tiling_on_sc=False),
    )
    def gather_k(table_hbm, idx_hbm, out_hbm, idx_v, rows_v, sem0, sem1):
        wid = lax.axis_index("s") * info.num_cores + lax.axis_index("c")
        base = wid * per_w
        sems = (sem0, sem1)

        def fire(g, buf):
            off = base + g * _CHUNK
            pltpu.sync_copy(idx_hbm.at[pl.ds(off // _IDXW, _GF)],
                            idx_v.at[buf])
            for j in range(_GF):
                pltpu.async_copy(
                    table_hbm.at[idx_v.at[buf].at[j]],
                    rows_v.at[buf].at[pl.ds(j * _IDXW, _IDXW)],
                    sems[buf],
                )

        def drain_wb(g, buf):
            for j in range(_GF):
                pltpu.make_async_copy(
                    table_hbm.at[idx_v.at[buf].at[j]],
                    rows_v.at[buf].at[pl.ds(j * _IDXW, _IDXW)],
                    sems[buf],
                ).wait()
            pltpu.sync_copy(rows_v.at[buf],
                            out_hbm.at[pl.ds(base + g * _CHUNK, _CHUNK)])

        fire(0, 0)

        def body(k, carry):
            g0 = 2 * k
            fire(g0 + 1, 1)
            drain_wb(g0, 0)

            @pl.when(k + 1 < n_pairs)
            def _():
                fire(g0 + 2, 0)

            drain_wb(g0 + 1, 1)
            return carry

        lax.fori_loop(0, n_pairs, body, 0)

    return gather_k


_TCOLS = 8192
_TH = _TCOLS // 2
VOCAB = 1000000
_NTBLK = (VOCAB + _TCOLS - 1) // _TCOLS
VPAD = _NTBLK * _TCOLS


def _repack_kernel(t_ref, o_ref):
    x = t_ref[...]
    o_ref[...] = jnp.concatenate([x[:, :_TH].T, x[:, _TH:].T], axis=1)


def _repack_table(tT):
    return pl.pallas_call(
        _repack_kernel,
        grid=(_NTBLK,),
        in_specs=[pl.BlockSpec((EMB, _TCOLS), lambda i: (0, i))],
        out_specs=pl.BlockSpec((_TH, 2 * EMB), lambda i: (i, 0)),
        out_shape=jax.ShapeDtypeStruct((VPAD // 2, 2 * EMB), jnp.float32),
    )(tT)


def _remap_idx(idx):
    blk = idx & (_TCOLS - 1)
    return (idx - blk) + ((blk & (_TH - 1)) << 1) + (blk >> 12)


_BP = 128
_NBLK = B // (2 * _BP)


def _dense_kernel(ea_ref, eb_ref, wp2_ref, bp2_ref, w1a_ref, w1b_ref, b1_ref,
                  w2t_ref, b2_ref, out_ref):
    wp2 = wp2_ref[...].astype(jnp.bfloat16)
    bp2 = bp2_ref[...]
    inv_l = 1.0 / L

    def tower(e_ref):
        e = e_ref[...].reshape(L * _BP, 2 * EMB).astype(jnp.bfloat16)
        p = jnp.dot(e, wp2, preferred_element_type=jnp.float32) + bp2
        p = jnp.maximum(p, 0.0)
        s = jnp.sum(p.reshape(L, _BP, 2 * PROJ), axis=0)
        return s[:, :PROJ] * inv_l, s[:, PROJ:] * inv_l

    maE, maO = tower(ea_ref)
    mbE, mbO = tower(eb_ref)
    w1a = w1a_ref[...].astype(jnp.bfloat16)
    w1b = w1b_ref[...].astype(jnp.bfloat16)

    def head(ma, mb, par):
        h = (jnp.dot(ma.astype(jnp.bfloat16), w1a,
                     preferred_element_type=jnp.float32)
             + jnp.dot(mb.astype(jnp.bfloat16), w1b,
                       preferred_element_type=jnp.float32)
             + b1_ref[...])
        h = jnp.maximum(h, 0.0)
        o = jnp.sum(h * w2t_ref[...], axis=1) + b2_ref[0]
        out_ref[0, par, :] = o

    head(maE, mbE, 0)
    head(maO, mbO, 1)


def kernel(a, b, table, W_proj, b_proj, W1, b1, W2, b2):
    n_tokens = 2 * B * L
    idx = _remap_idx(jnp.concatenate([a.T.reshape(-1), b.T.reshape(-1)]))
    idx2 = idx.reshape(n_tokens // _IDXW, _IDXW)
    table_lin = _repack_table(table.T).reshape(VPAD, EMB)
    gathered = _make_sc_gather(n_tokens)(table_lin, idx2)
    ev = gathered.reshape(2 * L, B // 2, 2 * EMB)

    wp2 = jnp.zeros((2 * EMB, 2 * PROJ), jnp.float32)
    wp2 = wp2.at[:EMB, :PROJ].set(W_proj).at[EMB:, PROJ:].set(W_proj)
    bp2 = jnp.concatenate([b_proj, b_proj]).reshape(1, 2 * PROJ)

    out = pl.pallas_call(
        _dense_kernel,
        grid=(_NBLK,),
        in_specs=[
            pl.BlockSpec((L, _BP, 2 * EMB), lambda i: (0, i, 0)),
            pl.BlockSpec((L, _BP, 2 * EMB), lambda i: (1, i, 0)),
            pl.BlockSpec((2 * EMB, 2 * PROJ), lambda i: (0, 0)),
            pl.BlockSpec((1, 2 * PROJ), lambda i: (0, 0)),
            pl.BlockSpec((PROJ, PROJ), lambda i: (0, 0)),
            pl.BlockSpec((PROJ, PROJ), lambda i: (0, 0)),
            pl.BlockSpec((1, PROJ), lambda i: (0, 0)),
            pl.BlockSpec((1, PROJ), lambda i: (0, 0)),
            pl.BlockSpec(memory_space=pltpu.SMEM),
        ],
        out_specs=pl.BlockSpec((1, 2, _BP), lambda i: (i, 0, 0)),
        out_shape=jax.ShapeDtypeStruct((_NBLK, 2, _BP), jnp.float32),
    )(
        ev, ev,
        wp2, bp2,
        W1[:PROJ], W1[PROJ:], b1.reshape(1, PROJ),
        W2.reshape(1, PROJ), b2,
    )
    return out.transpose(0, 2, 1).reshape(B)


# speedup vs baseline: 5.5039x; 1.0563x over previous
import functools

import jax
import jax.numpy as jnp
from jax import lax
from jax.experimental import pallas as pl
from jax.experimental.pallas import tpu as pltpu
from jax.experimental.pallas import tpu_sc as plsc

B = 16384
L = 50
EMB = 64
PROJ = 128

_IDXW = 128
_GF = 4
_CHUNK = _IDXW * _GF


def _make_sc_gather(n_tokens: int):
    info = plsc.get_sparse_core_info()
    nw = info.num_cores * info.num_subcores
    per_w = n_tokens // nw
    assert n_tokens % nw == 0 and per_w % (2 * _CHUNK) == 0
    n_pairs = per_w // (2 * _CHUNK)
    mesh = plsc.VectorSubcoreMesh(core_axis_name="c", subcore_axis_name="s")

    @functools.partial(
        pl.kernel,
        mesh=mesh,
        out_type=jax.ShapeDtypeStruct((n_tokens, EMB), jnp.float32),
        scratch_types=[
            pltpu.VMEM((2, _GF, _IDXW), jnp.int32),
            pltpu.VMEM((2, _CHUNK, EMB), jnp.float32),
            pltpu.SemaphoreType.DMA,
            pltpu.SemaphoreType.DMA,
        ],
        compiler_params=pltpu.CompilerParams(use_tc_tiling_on_sc=False),
    )
    def gather_k(table_hbm, idx_hbm, out_hbm, idx_v, rows_v, sem0, sem1):
        wid = lax.axis_index("s") * info.num_cores + lax.axis_index("c")
        base = wid * per_w
        sems = (sem0, sem1)

        def fire(g, buf):
            off = base + g * _CHUNK
            pltpu.sync_copy(idx_hbm.at[pl.ds(off // _IDXW, _GF)],
                            idx_v.at[buf])
            for j in range(_GF):
                pltpu.async_copy(
                    table_hbm.at[idx_v.at[buf].at[j]],
                    rows_v.at[buf].at[pl.ds(j * _IDXW, _IDXW)],
                    sems[buf],
                )

        def drain_wb(g, buf):
            for j in range(_GF):
                pltpu.make_async_copy(
                    table_hbm.at[idx_v.at[buf].at[j]],
                    rows_v.at[buf].at[pl.ds(j * _IDXW, _IDXW)],
                    sems[buf],
                ).wait()
            pltpu.sync_copy(rows_v.at[buf],
                            out_hbm.at[pl.ds(base + g * _CHUNK, _CHUNK)])

        fire(0, 0)

        def body(k, carry):
            g0 = 2 * k
            fire(g0 + 1, 1)
            drain_wb(g0, 0)

            @pl.when(k + 1 < n_pairs)
            def _():
                fire(g0 + 2, 0)

            drain_wb(g0 + 1, 1)
            return carry

        lax.fori_loop(0, n_pairs, body, 0)

    return gather_k


_TCOLS = 8192
_TH = _TCOLS // 2
VOCAB = 1000000
_NTBLK = (VOCAB + _TCOLS - 1) // _TCOLS
VPAD = _NTBLK * _TCOLS


def _repack_kernel(t_ref, o_ref):
    x = t_ref[...]
    o_ref[...] = jnp.concatenate([x[:, :_TH].T, x[:, _TH:].T], axis=1)


def _repack_table(tT):
    return pl.pallas_call(
        _repack_kernel,
        grid=(_NTBLK,),
        in_specs=[pl.BlockSpec((EMB, _TCOLS), lambda i: (0, i))],
        out_specs=pl.BlockSpec((_TH, 2 * EMB), lambda i: (i, 0)),
        out_shape=jax.ShapeDtypeStruct((VPAD // 2, 2 * EMB), jnp.float32),
    )(tT)


def _remap_idx(idx):
    blk = idx & (_TCOLS - 1)
    return (idx - blk) + ((blk & (_TH - 1)) << 1) + (blk >> 12)


_BP = 128
_NBLK = B // (2 * _BP)


def _dense_kernel(ea_ref, eb_ref, wp2_ref, bp2_ref, w1a_ref, w1b_ref, b1_ref,
                  w2t_ref, b2_ref, out_ref):
    wp2 = wp2_ref[...].astype(jnp.bfloat16)
    bp2 = bp2_ref[...]
    inv_l = 1.0 / L

    def tower(e_ref):
        e = e_ref[...].reshape(L * _BP, 2 * EMB).astype(jnp.bfloat16)
        p = jnp.dot(e, wp2, preferred_element_type=jnp.float32) + bp2
        p = jnp.maximum(p, 0.0)
        s = jnp.sum(p.reshape(L, _BP, 2 * PROJ), axis=0)
        return s[:, :PROJ] * inv_l, s[:, PROJ:] * inv_l

    maE, maO = tower(ea_ref)
    mbE, mbO = tower(eb_ref)
    w1a = w1a_ref[...].astype(jnp.bfloat16)
    w1b = w1b_ref[...].astype(jnp.bfloat16)

    def head(ma, mb, par):
        h = (jnp.dot(ma.astype(jnp.bfloat16), w1a,
                     preferred_element_type=jnp.float32)
             + jnp.dot(mb.astype(jnp.bfloat16), w1b,
                       preferred_element_type=jnp.float32)
             + b1_ref[...])
        h = jnp.maximum(h, 0.0)
        o = jnp.sum(h * w2t_ref[...], axis=1) + b2_ref[0]
        out_ref[0, par, :] = o

    head(maE, mbE, 0)
    head(maO, mbO, 1)


_NSEG = 2
_BSEG = B // _NSEG


def _dense_call(ev, weights, nblk):
    return pl.pallas_call(
        _dense_kernel,
        grid=(nblk,),
        in_specs=[
            pl.BlockSpec((L, _BP, 2 * EMB), lambda i: (0, i, 0)),
            pl.BlockSpec((L, _BP, 2 * EMB), lambda i: (1, i, 0)),
            pl.BlockSpec((2 * EMB, 2 * PROJ), lambda i: (0, 0)),
            pl.BlockSpec((1, 2 * PROJ), lambda i: (0, 0)),
            pl.BlockSpec((PROJ, PROJ), lambda i: (0, 0)),
            pl.BlockSpec((PROJ, PROJ), lambda i: (0, 0)),
            pl.BlockSpec((1, PROJ), lambda i: (0, 0)),
            pl.BlockSpec((1, PROJ), lambda i: (0, 0)),
            pl.BlockSpec(memory_space=pltpu.SMEM),
        ],
        out_specs=pl.BlockSpec((1, 2, _BP), lambda i: (i, 0, 0)),
        out_shape=jax.ShapeDtypeStruct((nblk, 2, _BP), jnp.float32),
    )(ev, ev, *weights)


def kernel(a, b, table, W_proj, b_proj, W1, b1, W2, b2):
    table_lin = _repack_table(table.T).reshape(VPAD, EMB)

    wp2 = jnp.zeros((2 * EMB, 2 * PROJ), jnp.float32)
    wp2 = wp2.at[:EMB, :PROJ].set(W_proj).at[EMB:, PROJ:].set(W_proj)
    bp2 = jnp.concatenate([b_proj, b_proj]).reshape(1, 2 * PROJ)
    weights = (wp2, bp2, W1[:PROJ], W1[PROJ:], b1.reshape(1, PROJ),
               W2.reshape(1, PROJ), b2)

    aT, bT = a.T, b.T
    n_tok_seg = 2 * _BSEG * L
    sc_gather = _make_sc_gather(n_tok_seg)
    nblk = _BSEG // (2 * _BP)
    outs = []
    for s in range(_NSEG):
        cols = slice(s * _BSEG, (s + 1) * _BSEG)
        idx = _remap_idx(jnp.concatenate(
            [aT[:, cols].reshape(-1), bT[:, cols].reshape(-1)]))
        idx2 = idx.reshape(n_tok_seg // _IDXW, _IDXW)
        gathered = sc_gather(table_lin, idx2)
        ev = gathered.reshape(2 * L, _BSEG // 2, 2 * EMB)
        outs.append(_dense_call(ev, weights, nblk))
    out = jnp.concatenate(outs, axis=0)
    return out.transpose(0, 2, 1).reshape(B)
